# Initial kernel scaffold; baseline (speedup 1.0000x reference)
import functools

import jax
import jax.numpy as jnp
from jax import lax
from jax.experimental import pallas as pl
from jax.experimental.pallas import tpu as pltpu
from jax.experimental.pallas import tpu_sc as plsc

_N_ENT = 10000
_N_USR = 10000
_D = 128
_NE = 320000
_NNZ = 160000
_NREL = 8
_NT = 16
_EB = 80
_ROWS_PER_TILE = _N_ENT // _NT
_EDGES_PER_TILE = _NE // _NT
_NNZ_PER_TILE = _NNZ // _NT

_mesh = plsc.VectorSubcoreMesh(core_axis_name="core", subcore_axis_name="subcore")


@functools.partial(
    pl.kernel,
    out_type=(
        jax.ShapeDtypeStruct((_N_ENT, _D), jnp.float32),
        jax.ShapeDtypeStruct((_N_USR, _D), jnp.float32),
        jax.ShapeDtypeStruct((_N_ENT, 16), jnp.float32),
    ),
    mesh=_mesh,
    scratch_types=[
        pltpu.VMEM_SHARED((_N_ENT, _D), jnp.float32),
        pltpu.VMEM_SHARED((_N_ENT, 16), jnp.float32),
        pltpu.VMEM((_EB,), jnp.int32),
        pltpu.VMEM((_EB,), jnp.int32),
        pltpu.VMEM((_EB,), jnp.int32),
        pltpu.VMEM((_EB,), jnp.float32),
        pltpu.VMEM((_EB, _D), jnp.float32),
        pltpu.VMEM((_NREL, _D), jnp.float32),
        pltpu.VMEM((_EB, 16), jnp.float32),
        pltpu.SemaphoreType.DMA,
    ],
)
def _sc_agg(ent_hbm, head_hbm, tail_hbm, rel_hbm, urow_hbm, ucol_hbm,
            uval_hbm, w_hbm, z128_hbm, z16_hbm, o16_hbm,
            esum_hbm, usum_hbm, cnt_hbm,
            sums, cnt, idx_g, idx_s, rel_buf, val_buf, data_buf, wbuf,
            ones_buf, sem):
  tid = lax.axis_index("subcore")
  cid = lax.axis_index("core")
  iota16 = lax.iota(jnp.int32, 16)

  r0 = tid * _ROWS_PER_TILE
  pltpu.sync_copy(z128_hbm.at[pl.ds(r0, _ROWS_PER_TILE)],
                  sums.at[pl.ds(r0, _ROWS_PER_TILE)])

  @pl.when(cid == 1)
  def _zero_cnt():
    pltpu.sync_copy(z16_hbm.at[pl.ds(r0, _ROWS_PER_TILE)],
                    cnt.at[pl.ds(r0, _ROWS_PER_TILE)])

  plsc.subcore_barrier()

  @pl.when(cid == 0)
  def _edges():
    pltpu.sync_copy(w_hbm, wbuf)
    ebase = tid * _EDGES_PER_TILE

    @pl.loop(0, _EDGES_PER_TILE, step=_EB)
    def _(b0):
      pltpu.sync_copy(tail_hbm.at[pl.ds(ebase + b0, _EB)], idx_g)
      pltpu.sync_copy(rel_hbm.at[pl.ds(ebase + b0, _EB)], rel_buf)
      pltpu.sync_copy(head_hbm.at[pl.ds(ebase + b0, _EB)], idx_s)
      pltpu.async_copy(ent_hbm.at[idx_g], data_buf, sem).wait()

      @pl.loop(0, _EB, step=16)
      def _(g):
        relv = rel_buf[pl.ds(g, 16)]
        for j in range(16):
          rb = jnp.take(relv, jnp.full((16,), j, jnp.int32),
                        mode="promise_in_bounds")
          e = g + j
          for k in range(_D // 16):
            wk = plsc.load_gather(wbuf, [rb, iota16 + (16 * k)])
            data_buf[e, pl.ds(16 * k, 16)] = (
                data_buf[e, pl.ds(16 * k, 16)] * wk)

      pltpu.sync_copy(data_buf, sums.at[idx_s], add=True)

  @pl.when(cid == 1)
  def _users():
    pltpu.sync_copy(o16_hbm, ones_buf)
    nbase = tid * _NNZ_PER_TILE

    @pl.loop(0, _NNZ_PER_TILE, step=_EB)
    def _(b0):
      pltpu.sync_copy(ucol_hbm.at[pl.ds(nbase + b0, _EB)], idx_g)
      pltpu.sync_copy(uval_hbm.at[pl.ds(nbase + b0, _EB)], val_buf)
      pltpu.sync_copy(urow_hbm.at[pl.ds(nbase + b0, _EB)], idx_s)
      pltpu.async_copy(ent_hbm.at[idx_g], data_buf, sem).wait()

      @pl.loop(0, _EB, step=16)
      def _(g):
        valv = val_buf[pl.ds(g, 16)]
        for j in range(16):
          vb = jnp.take(valv, jnp.full((16,), j, jnp.int32),
                        mode="promise_in_bounds")
          e = g + j
          for k in range(_D // 16):
            data_buf[e, pl.ds(16 * k, 16)] = (
                data_buf[e, pl.ds(16 * k, 16)] * vb)

      pltpu.sync_copy(data_buf, sums.at[idx_s], add=True)

    ebase = tid * _EDGES_PER_TILE

    @pl.loop(0, _EDGES_PER_TILE, step=_EB)
    def _(b0):
      pltpu.sync_copy(head_hbm.at[pl.ds(ebase + b0, _EB)], idx_s)
      pltpu.sync_copy(ones_buf, cnt.at[idx_s], add=True)

  plsc.subcore_barrier()

  @pl.when(cid == 0)
  def _out_e():
    pltpu.sync_copy(sums.at[pl.ds(r0, _ROWS_PER_TILE)],
                    esum_hbm.at[pl.ds(r0, _ROWS_PER_TILE)])

  @pl.when(cid == 1)
  def _out_u():
    pltpu.sync_copy(sums.at[pl.ds(r0, _ROWS_PER_TILE)],
                    usum_hbm.at[pl.ds(r0, _ROWS_PER_TILE)])
    pltpu.sync_copy(cnt.at[pl.ds(r0, _ROWS_PER_TILE)],
                    cnt_hbm.at[pl.ds(r0, _ROWS_PER_TILE)])


def _finish_body(esum_ref, cnt_ref, usum_ref, user_ref, lat_ref, w_ref,
                 datt_ref, eout_ref, uout_ref):
  c = cnt_ref[:, 0:1]
  eout_ref[...] = esum_ref[...] / jnp.maximum(c, 1.0)
  logits = lax.dot_general(
      user_ref[...], lat_ref[...], (((1,), (1,)), ((), ())),
      precision=lax.Precision.HIGHEST, preferred_element_type=jnp.float32)
  score = jax.nn.softmax(logits, axis=1)
  dw = jax.nn.softmax(datt_ref[...], axis=-1)
  dw2 = lax.dot_general(
      dw, w_ref[...], (((1,), (0,)), ((), ())),
      precision=lax.Precision.HIGHEST, preferred_element_type=jnp.float32)
  mod = lax.dot_general(
      score, dw2, (((1,), (0,)), ((), ())),
      precision=lax.Precision.HIGHEST, preferred_element_type=jnp.float32)
  uout_ref[...] = usum_ref[...] * (1.0 + mod)


_BLK = 1000


def _finish(esum, cnt, usum, user_emb, latent_emb, weight, disen_weight_att):
  n_blocks = _N_USR // _BLK
  return pl.pallas_call(
      _finish_body,
      grid=(n_blocks,),
      in_specs=[
          pl.BlockSpec((_BLK, _D), lambda i: (i, 0)),
          pl.BlockSpec((_BLK, 16), lambda i: (i, 0)),
          pl.BlockSpec((_BLK, _D), lambda i: (i, 0)),
          pl.BlockSpec((_BLK, _D), lambda i: (i, 0)),
          pl.BlockSpec((4, _D), lambda i: (0, 0)),
          pl.BlockSpec((_NREL, _D), lambda i: (0, 0)),
          pl.BlockSpec((4, _NREL), lambda i: (0, 0)),
      ],
      out_specs=[
          pl.BlockSpec((_BLK, _D), lambda i: (i, 0)),
          pl.BlockSpec((_BLK, _D), lambda i: (i, 0)),
      ],
      out_shape=(
          jax.ShapeDtypeStruct((_N_ENT, _D), jnp.float32),
          jax.ShapeDtypeStruct((_N_USR, _D), jnp.float32),
      ),
  )(esum, cnt, usum, user_emb, latent_emb, weight, disen_weight_att)


def kernel(entity_emb, user_emb, latent_emb, edge_index, edge_type,
           interact_idx, interact_val, weight, disen_weight_att):
  head = edge_index[0].astype(jnp.int32)
  tail = edge_index[1].astype(jnp.int32)
  rel = ((edge_type.astype(jnp.int32) - 1) % _NREL).astype(jnp.int32)
  urow = interact_idx[0].astype(jnp.int32)
  ucol = interact_idx[1].astype(jnp.int32)
  z128 = jnp.zeros((_N_ENT, _D), jnp.float32)
  z16 = jnp.zeros((_N_ENT, 16), jnp.float32)
  o16 = jnp.ones((_EB, 16), jnp.float32)
  esum, usum, cnt = _sc_agg(entity_emb, head, tail, rel, urow, ucol,
                            interact_val, weight, z128, z16, o16)
  return _finish(esum, cnt, usum, user_emb, latent_emb, weight,
                 disen_weight_att)


# speedup vs baseline: 2.2534x; 2.2534x over previous
import dataclasses
import functools

import jax
import jax.numpy as jnp
from jax import lax
from jax.experimental import pallas as pl
from jax.experimental.pallas import tpu as pltpu
from jax.experimental.pallas import tpu_sc as plsc

_N_ENT = 10000
_N_USR = 10000
_D = 128
_NE = 320000
_NNZ = 160000
_NREL = 8
_NT = 16
_EB = 80
_RPT = 624
_TAIL0 = _RPT * _NT
_TAILN = _N_ENT - _TAIL0
_CHK = 48
_EDGES_PER_TILE = _NE // _NT
_NNZ_PER_TILE = _NNZ // _NT

_mesh = plsc.VectorSubcoreMesh(core_axis_name="core", subcore_axis_name="subcore")

_cp = pltpu.CompilerParams()
if "needs_layout_passes" in pltpu.CompilerParams.__dataclass_fields__:
  _cp = dataclasses.replace(_cp, needs_layout_passes=False)

_GDN = lax.GatherDimensionNumbers(
    offset_dims=(), collapsed_slice_dims=(0,), start_index_map=(0,))


def _vtake(vec, idx):
  return lax.gather(vec, idx[:, None], _GDN, slice_sizes=(1,),
                    mode=lax.GatherScatterMode.PROMISE_IN_BOUNDS)


def _copy_rows(src, dst, tid, bounce):
  r0 = tid * _RPT

  @pl.loop(0, _RPT, step=_CHK)
  def _(c):
    pltpu.sync_copy(src.at[pl.ds(r0 + c, _CHK)], bounce.at[pl.ds(0, _CHK)])
    pltpu.sync_copy(bounce.at[pl.ds(0, _CHK)], dst.at[pl.ds(r0 + c, _CHK)])

  @pl.when(tid == _NT - 1)
  def _():
    pltpu.sync_copy(src.at[pl.ds(_TAIL0, _TAILN)], bounce.at[pl.ds(0, _TAILN)])
    pltpu.sync_copy(bounce.at[pl.ds(0, _TAILN)], dst.at[pl.ds(_TAIL0, _TAILN)])


@functools.partial(
    pl.kernel,
    out_type=(
        jax.ShapeDtypeStruct((_N_ENT, _D), jnp.float32),
        jax.ShapeDtypeStruct((_N_USR, _D), jnp.float32),
    ),
    mesh=_mesh,
    compiler_params=_cp,
    scratch_types=[
        pltpu.VMEM_SHARED((_N_ENT, _D), jnp.float32),
        pltpu.VMEM_SHARED((_N_ENT,), jnp.float32),
        pltpu.VMEM((_EB,), jnp.int32),
        pltpu.VMEM((_EB,), jnp.int32),
        pltpu.VMEM((_EB,), jnp.int32),
        pltpu.VMEM((_EB,), jnp.float32),
        pltpu.VMEM((_EB, _D), jnp.float32),
        pltpu.VMEM((_NREL, _D), jnp.float32),
        pltpu.VMEM((_EB,), jnp.float32),
        pltpu.VMEM((_CHK,), jnp.float32),
        pltpu.SemaphoreType.DMA,
    ],
)
def _sc_agg(ent_hbm, head_hbm, tail_hbm, rel_hbm, urow_hbm, ucol_hbm,
            uval_hbm, w_hbm, z128_hbm,
            eagg_hbm, usum_hbm,
            sums, cnt, idx_g, idx_s, rel_buf, val_buf, data_buf, wbuf,
            ones_v, cnt_v, sem):
  tid = lax.axis_index("subcore")
  cid = lax.axis_index("core")
  r0 = tid * _RPT

  _copy_rows(z128_hbm, sums, tid, data_buf)

  @pl.when(cid == 0)
  def _zero_cnt():
    @pl.loop(0, _CHK, step=16)
    def _(i):
      cnt_v[pl.ds(i, 16)] = jnp.zeros((16,), jnp.float32)

    @pl.loop(0, _RPT, step=_CHK)
    def _(c):
      pltpu.sync_copy(cnt_v, cnt.at[pl.ds(r0 + c, _CHK)])

    @pl.when(tid == _NT - 1)
    def _():
      pltpu.sync_copy(cnt_v.at[pl.ds(0, _TAILN)],
                      cnt.at[pl.ds(_TAIL0, _TAILN)])

    @pl.loop(0, _EB, step=16)
    def _(i):
      ones_v[pl.ds(i, 16)] = jnp.ones((16,), jnp.float32)

  plsc.subcore_barrier()

  @pl.when(cid == 0)
  def _edges():
    pltpu.sync_copy(w_hbm, wbuf)
    ebase = tid * _EDGES_PER_TILE

    @pl.loop(0, _EDGES_PER_TILE, step=_EB)
    def _(b0):
      pltpu.sync_copy(tail_hbm.at[pl.ds(ebase + b0, _EB)], idx_g)
      pltpu.sync_copy(rel_hbm.at[pl.ds(ebase + b0, _EB)], rel_buf)
      pltpu.sync_copy(head_hbm.at[pl.ds(ebase + b0, _EB)], idx_s)
      pltpu.async_copy(ent_hbm.at[idx_g], data_buf, sem).wait()

      @pl.loop(0, _EB, step=16)
      def _(g):
        relv = rel_buf[pl.ds(g, 16)]
        iota16 = lax.iota(jnp.int32, 16)
        for j in range(16):
          rb = _vtake(relv, jnp.full((16,), j, jnp.int32))
          e = g + j
          for k in range(_D // 16):
            wk = plsc.load_gather(wbuf, [rb, iota16 + (16 * k)])
            data_buf[e, pl.ds(16 * k, 16)] = (
                data_buf[e, pl.ds(16 * k, 16)] * wk)

      pltpu.sync_copy(data_buf, sums.at[idx_s], add=True)
      pltpu.sync_copy(ones_v, cnt.at[idx_s], add=True)

  @pl.when(cid == 1)
  def _users():
    nbase = tid * _NNZ_PER_TILE

    @pl.loop(0, _NNZ_PER_TILE, step=_EB)
    def _(b0):
      pltpu.sync_copy(ucol_hbm.at[pl.ds(nbase + b0, _EB)], idx_g)
      pltpu.sync_copy(uval_hbm.at[pl.ds(nbase + b0, _EB)], val_buf)
      pltpu.sync_copy(urow_hbm.at[pl.ds(nbase + b0, _EB)], idx_s)
      pltpu.async_copy(ent_hbm.at[idx_g], data_buf, sem).wait()

      @pl.loop(0, _EB, step=16)
      def _(g):
        valv = val_buf[pl.ds(g, 16)]
        for j in range(16):
          vb = _vtake(valv, jnp.full((16,), j, jnp.int32))
          e = g + j
          for k in range(_D // 16):
            data_buf[e, pl.ds(16 * k, 16)] = (
                data_buf[e, pl.ds(16 * k, 16)] * vb)

      pltpu.sync_copy(data_buf, sums.at[idx_s], add=True)

  plsc.subcore_barrier()

  @pl.when(cid == 0)
  def _out_e():
    def _emit(c, n):
      pltpu.sync_copy(sums.at[pl.ds(r0 + c, n)], data_buf.at[pl.ds(0, n)])
      pltpu.sync_copy(cnt.at[pl.ds(r0 + c, n)], cnt_v.at[pl.ds(0, n)])

      @pl.loop(0, n, step=16)
      def _(g):
        cv = cnt_v[pl.ds(g, 16)]
        rv = 1.0 / jnp.maximum(cv, 1.0)
        for j in range(16):
          sb = _vtake(rv, jnp.full((16,), j, jnp.int32))
          e = g + j
          for k in range(_D // 16):
            data_buf[e, pl.ds(16 * k, 16)] = (
                data_buf[e, pl.ds(16 * k, 16)] * sb)

      pltpu.sync_copy(data_buf.at[pl.ds(0, n)], eagg_hbm.at[pl.ds(r0 + c, n)])

    @pl.loop(0, _RPT, step=_CHK)
    def _(c):
      _emit(c, _CHK)

    @pl.when(tid == _NT - 1)
    def _():
      _emit(_TAIL0 - r0, _TAILN)

  @pl.when(cid == 1)
  def _out_u():
    _copy_rows(sums, usum_hbm, tid, data_buf)


def _finish_body(usum_ref, user_ref, lat_ref, w_ref, datt_ref, uout_ref):
  logits = lax.dot_general(
      user_ref[...], lat_ref[...], (((1,), (1,)), ((), ())),
      precision=lax.Precision.HIGHEST, preferred_element_type=jnp.float32)
  score = jax.nn.softmax(logits, axis=1)
  dw = jax.nn.softmax(datt_ref[...], axis=-1)
  dw2 = lax.dot_general(
      dw, w_ref[...], (((1,), (0,)), ((), ())),
      precision=lax.Precision.HIGHEST, preferred_element_type=jnp.float32)
  mod = lax.dot_general(
      score, dw2, (((1,), (0,)), ((), ())),
      precision=lax.Precision.HIGHEST, preferred_element_type=jnp.float32)
  uout_ref[...] = usum_ref[...] * (1.0 + mod)


_BLK = 1000


def _finish(usum, user_emb, latent_emb, weight, disen_weight_att):
  n_blocks = _N_USR // _BLK
  return pl.pallas_call(
      _finish_body,
      grid=(n_blocks,),
      in_specs=[
          pl.BlockSpec((_BLK, _D), lambda i: (i, 0)),
          pl.BlockSpec((_BLK, _D), lambda i: (i, 0)),
          pl.BlockSpec((4, _D), lambda i: (0, 0)),
          pl.BlockSpec((_NREL, _D), lambda i: (0, 0)),
          pl.BlockSpec((4, _NREL), lambda i: (0, 0)),
      ],
      out_specs=pl.BlockSpec((_BLK, _D), lambda i: (i, 0)),
      out_shape=jax.ShapeDtypeStruct((_N_USR, _D), jnp.float32),
  )(usum, user_emb, latent_emb, weight, disen_weight_att)


def kernel(entity_emb, user_emb, latent_emb, edge_index, edge_type,
           interact_idx, interact_val, weight, disen_weight_att):
  head = edge_index[0].astype(jnp.int32)
  tail = edge_index[1].astype(jnp.int32)
  rel = ((edge_type.astype(jnp.int32) - 1) % _NREL).astype(jnp.int32)
  urow = interact_idx[0].astype(jnp.int32)
  ucol = interact_idx[1].astype(jnp.int32)
  z128 = jnp.zeros((_N_ENT, _D), jnp.float32)
  eagg, usum = _sc_agg(entity_emb, head, tail, rel, urow, ucol,
                       interact_val, weight, z128)
  user_agg = _finish(usum, user_emb, latent_emb, weight, disen_weight_att)
  return (eagg, user_agg)


# speedup vs baseline: 4.0713x; 1.8067x over previous
import dataclasses
import functools

import jax
import jax.numpy as jnp
from jax import lax
from jax.experimental import pallas as pl
from jax.experimental.pallas import tpu as pltpu
from jax.experimental.pallas import tpu_sc as plsc

_N_ENT = 10000
_N_USR = 10000
_D = 128
_NE = 320000
_NNZ = 160000
_NREL = 8
_NT = 16
_EB = 80
_RPT = 624
_TAIL0 = _RPT * _NT
_TAILN = _N_ENT - _TAIL0
_EDGES_PER_TILE = _NE // _NT
_NNZ_PER_TILE = _NNZ // _NT
_NB_E = _EDGES_PER_TILE // _EB
_NB_U = _NNZ_PER_TILE // _EB

_mesh = plsc.VectorSubcoreMesh(core_axis_name="core", subcore_axis_name="subcore")

_cp = pltpu.CompilerParams()
if "needs_layout_passes" in pltpu.CompilerParams.__dataclass_fields__:
  _cp = dataclasses.replace(_cp, needs_layout_passes=False)

_GDN = lax.GatherDimensionNumbers(
    offset_dims=(), collapsed_slice_dims=(0,), start_index_map=(0,))


def _vtake(vec, idx):
  return lax.gather(vec, idx[:, None], _GDN, slice_sizes=(1,),
                    mode=lax.GatherScatterMode.PROMISE_IN_BOUNDS)


@functools.partial(
    pl.kernel,
    out_type=(
        jax.ShapeDtypeStruct((_N_ENT, _D), jnp.float32),
        jax.ShapeDtypeStruct((_N_USR, _D), jnp.float32),
    ),
    mesh=_mesh,
    compiler_params=_cp,
    scratch_types=[
        pltpu.VMEM_SHARED((_N_ENT, _D), jnp.float32),
        pltpu.VMEM_SHARED((_N_ENT,), jnp.float32),
        pltpu.VMEM((_EB, _D), jnp.float32),
        pltpu.VMEM((_EB, _D), jnp.float32),
        pltpu.VMEM((_EB, _D), jnp.float32),
        pltpu.VMEM((_EB, _D), jnp.float32),
        pltpu.VMEM((_EB,), jnp.int32),
        pltpu.VMEM((_EB,), jnp.int32),
        pltpu.VMEM((_EB,), jnp.int32),
        pltpu.VMEM((_EB,), jnp.int32),
        pltpu.VMEM((_EB,), jnp.int32),
        pltpu.VMEM((_EB,), jnp.int32),
        pltpu.VMEM((_EB,), jnp.int32),
        pltpu.VMEM((_EB,), jnp.int32),
        pltpu.VMEM((_EB,), jnp.int32),
        pltpu.VMEM((_EB,), jnp.int32),
        pltpu.VMEM((_EB,), jnp.int32),
        pltpu.VMEM((_EB,), jnp.int32),
        pltpu.VMEM((_NREL, _D), jnp.float32),
        pltpu.VMEM((_EB,), jnp.float32),
        pltpu.VMEM((_EB,), jnp.float32),
    ] + [pltpu.SemaphoreType.DMA] * 12,
)
def _sc_agg(ent_hbm, head_hbm, tail_hbm, rel_hbm, urow_hbm, ucol_hbm,
            uval_hbm, w_hbm,
            eagg_hbm, usum_hbm,
            sums, cnt, d0, d1, d2, d3, t0, t1, t2, t3, h0, h1, h2, h3,
            a0, a1, a2, a3, wbuf, ones_v, cz_v,
            si0, si1, si2, si3, sg0, sg1, sg2, sg3, ss0, ss1, ss2, ss3):
  tid = lax.axis_index("subcore")
  cid = lax.axis_index("core")
  r0 = tid * _RPT
  ds_ = (d0, d1, d2, d3)
  ts_ = (t0, t1, t2, t3)
  hs_ = (h0, h1, h2, h3)
  as_ = (a0, a1, a2, a3)
  isem = (si0, si1, si2, si3)
  gsem = (sg0, sg1, sg2, sg3)
  ssem = (ss0, ss1, ss2, ss3)

  @pl.loop(0, _EB, step=16)
  def _(i):
    cz_v[pl.ds(i, 16)] = jnp.zeros((16,), jnp.float32)
    ones_v[pl.ds(i, 16)] = jnp.ones((16,), jnp.float32)

  @pl.loop(0, _EB)
  def _(i):
    for k in range(_D // 16):
      d0[i, pl.ds(16 * k, 16)] = jnp.zeros((16,), jnp.float32)

  @pl.loop(0, 560, step=80)
  def _(c):
    pltpu.async_copy(d0, sums.at[pl.ds(r0 + c, _EB)], sg0)

  pltpu.async_copy(d0.at[pl.ds(0, 64)], sums.at[pl.ds(r0 + 560, 64)], sg1)

  @pl.when(cid == 0)
  def _zcnt():
    @pl.loop(0, 560, step=80)
    def _(c):
      pltpu.async_copy(cz_v, cnt.at[pl.ds(r0 + c, _EB)], sg2)

    pltpu.async_copy(cz_v.at[pl.ds(0, 64)], cnt.at[pl.ds(r0 + 560, 64)], sg3)

    @pl.when(tid == _NT - 1)
    def _():
      pltpu.sync_copy(cz_v.at[pl.ds(0, _TAILN)], cnt.at[pl.ds(_TAIL0, _TAILN)])

  @pl.when(tid == _NT - 1)
  def _():
    pltpu.sync_copy(d0.at[pl.ds(0, _TAILN)], sums.at[pl.ds(_TAIL0, _TAILN)])

  @pl.loop(0, 560, step=80)
  def _(c):
    pltpu.make_async_copy(d0, sums.at[pl.ds(r0 + c, _EB)], sg0).wait()

  pltpu.make_async_copy(d0.at[pl.ds(0, 64)], sums.at[pl.ds(r0 + 560, 64)],
                        sg1).wait()

  @pl.when(cid == 0)
  def _zcnt_wait():
    @pl.loop(0, 560, step=80)
    def _(c):
      pltpu.make_async_copy(cz_v, cnt.at[pl.ds(r0 + c, _EB)], sg2).wait()

    pltpu.make_async_copy(cz_v.at[pl.ds(0, 64)], cnt.at[pl.ds(r0 + 560, 64)],
                          sg3).wait()

  plsc.subcore_barrier()

  def _pump(nb, gidx_hbm, sidx_hbm, aux_hbm, base, mul_fn, with_cnt):
    def _issue_idx(b, u):
      off = pl.ds(base + b * _EB, _EB)
      pltpu.async_copy(gidx_hbm.at[off], ts_[u], isem[u])
      pltpu.async_copy(sidx_hbm.at[off], hs_[u], isem[u])
      if aux_hbm is not None:
        pltpu.async_copy(aux_hbm.at[off], as_[u], isem[u])

    def _wait_idx(b, u):
      off = pl.ds(base + b * _EB, _EB)
      pltpu.make_async_copy(gidx_hbm.at[off], ts_[u], isem[u]).wait()
      pltpu.make_async_copy(sidx_hbm.at[off], hs_[u], isem[u]).wait()
      if aux_hbm is not None:
        pltpu.make_async_copy(aux_hbm.at[off], as_[u], isem[u]).wait()

    def _issue_gather(u):
      pltpu.async_copy(ent_hbm.at[ts_[u]], ds_[u], gsem[u])

    def _wait_gather(u):
      pltpu.make_async_copy(ent_hbm.at[ts_[u]], ds_[u], gsem[u]).wait()

    def _issue_scat(u):
      pltpu.async_copy(ds_[u], sums.at[hs_[u]], ssem[u], add=True)
      if with_cnt:
        pltpu.async_copy(ones_v, cnt.at[hs_[u]], ssem[u], add=True)

    def _wait_scat(u):
      pltpu.make_async_copy(ds_[u], sums.at[hs_[u]], ssem[u]).wait()
      if with_cnt:
        pltpu.make_async_copy(ones_v, cnt.at[hs_[u]], ssem[u]).wait()

    def _maybe(cond, fn):
      if isinstance(cond, bool):
        if cond:
          fn()
      else:
        @pl.when(cond)
        def _():
          fn()

    def _section(b, u):
      _maybe(b + 1 < nb, lambda: _wait_idx(b + 1, (u + 1) % 4))
      _maybe(b >= 2, lambda: _wait_scat((u + 2) % 4))
      _maybe(b + 1 < nb, lambda: _issue_gather((u + 1) % 4))
      _maybe(b + 2 < nb, lambda: _issue_idx(b + 2, (u + 2) % 4))
      _wait_gather(u)
      mul_fn(ds_[u], as_[u], b)
      _issue_scat(u)

    _issue_idx(0, 0)
    _issue_idx(1, 1)
    _wait_idx(0, 0)
    _issue_gather(0)

    nb4 = nb - (nb % 4)

    @pl.loop(0, nb4, step=4)
    def _(b0):
      for u in range(4):
        _section(b0 + u, u)

    for t in range(nb % 4):
      _section(nb4 + t, t)

    _wait_scat((nb - 2) % 4)
    _wait_scat((nb - 1) % 4)

  def _mul_edges(dbuf, abuf, b):
    iota16 = lax.iota(jnp.int32, 16)

    @pl.loop(0, _EB, step=16)
    def _(g):
      relv = abuf[pl.ds(g, 16)]
      for j in range(16):
        rb = _vtake(relv, jnp.full((16,), j, jnp.int32))
        e = g + j
        for k in range(_D // 16):
          wk = plsc.load_gather(wbuf, [rb, iota16 + (16 * k)])
          dbuf[e, pl.ds(16 * k, 16)] = dbuf[e, pl.ds(16 * k, 16)] * wk

  def _mul_users(dbuf, abuf, b):
    @pl.loop(0, _EB, step=16)
    def _(g):
      valv = plsc.bitcast(abuf[pl.ds(g, 16)], jnp.float32)
      for j in range(16):
        vb = _vtake(valv, jnp.full((16,), j, jnp.int32))
        e = g + j
        for k in range(_D // 16):
          dbuf[e, pl.ds(16 * k, 16)] = dbuf[e, pl.ds(16 * k, 16)] * vb

  @pl.when(cid == 0)
  def _edges():
    pltpu.sync_copy(w_hbm, wbuf)
    _pump(_NB_E, tail_hbm, head_hbm, rel_hbm, tid * _EDGES_PER_TILE,
          _mul_edges, with_cnt=True)

  @pl.when(cid == 1)
  def _users():
    _pump(_NB_U, ucol_hbm, urow_hbm, uval_hbm, tid * _NNZ_PER_TILE,
          _mul_users, with_cnt=False)

  plsc.subcore_barrier()

  def _divide(dbuf, n):
    @pl.loop(0, n, step=16)
    def _(g):
      cv = cz_v[pl.ds(g, 16)]
      rv = 1.0 / jnp.maximum(cv, 1.0)
      for j in range(16):
        sb = _vtake(rv, jnp.full((16,), j, jnp.int32))
        e = g + j
        for k in range(_D // 16):
          dbuf[e, pl.ds(16 * k, 16)] = dbuf[e, pl.ds(16 * k, 16)] * sb

  def _copy_out(dst_hbm, divide):
    def _chunk(c, n, u):
      pltpu.sync_copy(sums.at[pl.ds(r0 + c, n)], ds_[u].at[pl.ds(0, n)])
      if divide:
        pltpu.sync_copy(cnt.at[pl.ds(r0 + c, n)], cz_v.at[pl.ds(0, n)])
        _divide(ds_[u], n)
      pltpu.async_copy(ds_[u].at[pl.ds(0, n)], dst_hbm.at[pl.ds(r0 + c, n)],
                       gsem[u])

    for ci in range(8):
      u = ci % 4
      n = _EB if ci < 7 else 64
      if ci >= 4:
        pltpu.make_async_copy(ds_[u].at[pl.ds(0, _EB)],
                              dst_hbm.at[pl.ds(r0 + (ci - 4) * _EB, _EB)],
                              gsem[u]).wait()
      _chunk(ci * _EB, n, u)

    for ci in range(4, 8):
      u = ci % 4
      n = _EB if ci < 7 else 64
      pltpu.make_async_copy(ds_[u].at[pl.ds(0, n)],
                            dst_hbm.at[pl.ds(r0 + ci * _EB, n)],
                            gsem[u]).wait()

    @pl.when(tid == _NT - 1)
    def _():
      pltpu.sync_copy(sums.at[pl.ds(_TAIL0, _TAILN)],
                      ds_[0].at[pl.ds(0, _TAILN)])
      if divide:
        pltpu.sync_copy(cnt.at[pl.ds(_TAIL0, _TAILN)],
                        cz_v.at[pl.ds(0, _TAILN)])
        _divide(ds_[0], _TAILN)
      pltpu.sync_copy(ds_[0].at[pl.ds(0, _TAILN)],
                      dst_hbm.at[pl.ds(_TAIL0, _TAILN)])

  @pl.when(cid == 0)
  def _out_e():
    _copy_out(eagg_hbm, divide=True)

  @pl.when(cid == 1)
  def _out_u():
    _copy_out(usum_hbm, divide=False)


def _finish_body(usum_ref, user_ref, lat_ref, w_ref, datt_ref, uout_ref):
  logits = lax.dot_general(
      user_ref[...], lat_ref[...], (((1,), (1,)), ((), ())),
      precision=lax.Precision.HIGHEST, preferred_element_type=jnp.float32)
  score = jax.nn.softmax(logits, axis=1)
  dw = jax.nn.softmax(datt_ref[...], axis=-1)
  dw2 = lax.dot_general(
      dw, w_ref[...], (((1,), (0,)), ((), ())),
      precision=lax.Precision.HIGHEST, preferred_element_type=jnp.float32)
  mod = lax.dot_general(
      score, dw2, (((1,), (0,)), ((), ())),
      precision=lax.Precision.HIGHEST, preferred_element_type=jnp.float32)
  uout_ref[...] = usum_ref[...] * (1.0 + mod)


_BLK = 1000


def _finish(usum, user_emb, latent_emb, weight, disen_weight_att):
  n_blocks = _N_USR // _BLK
  return pl.pallas_call(
      _finish_body,
      grid=(n_blocks,),
      in_specs=[
          pl.BlockSpec((_BLK, _D), lambda i: (i, 0)),
          pl.BlockSpec((_BLK, _D), lambda i: (i, 0)),
          pl.BlockSpec((4, _D), lambda i: (0, 0)),
          pl.BlockSpec((_NREL, _D), lambda i: (0, 0)),
          pl.BlockSpec((4, _NREL), lambda i: (0, 0)),
      ],
      out_specs=pl.BlockSpec((_BLK, _D), lambda i: (i, 0)),
      out_shape=jax.ShapeDtypeStruct((_N_USR, _D), jnp.float32),
  )(usum, user_emb, latent_emb, weight, disen_weight_att)


def kernel(entity_emb, user_emb, latent_emb, edge_index, edge_type,
           interact_idx, interact_val, weight, disen_weight_att):
  head = edge_index[0].astype(jnp.int32)
  tail = edge_index[1].astype(jnp.int32)
  rel = ((edge_type.astype(jnp.int32) - 1) % _NREL).astype(jnp.int32)
  urow = interact_idx[0].astype(jnp.int32)
  ucol = interact_idx[1].astype(jnp.int32)
  uval_i = lax.bitcast_convert_type(interact_val, jnp.int32)
  eagg, usum = _sc_agg(entity_emb, head, tail, rel, urow, ucol,
                       uval_i, weight)
  user_agg = _finish(usum, user_emb, latent_emb, weight, disen_weight_att)
  return (eagg, user_agg)


# speedup vs baseline: 11.0400x; 2.7116x over previous
import dataclasses
import functools

import jax
import jax.numpy as jnp
from jax import lax
from jax.experimental import pallas as pl
from jax.experimental.pallas import tpu as pltpu
from jax.experimental.pallas import tpu_sc as plsc

_N_ENT = 10000
_N_USR = 10000
_D = 128
_NE = 320000
_NNZ = 160000
_NREL = 8
_NT = 16
_EB = 80
_RPT = 624
_TAIL0 = _RPT * _NT
_TAILN = _N_ENT - _TAIL0
_EDGES_PER_TILE = _NE // _NT
_NNZ_PER_TILE = _NNZ // _NT
_NB_E = _EDGES_PER_TILE // _EB
_NB_U = _NNZ_PER_TILE // _EB

_mesh = plsc.VectorSubcoreMesh(core_axis_name="core", subcore_axis_name="subcore")

_cp = pltpu.CompilerParams()
if "needs_layout_passes" in pltpu.CompilerParams.__dataclass_fields__:
  _cp = dataclasses.replace(_cp, needs_layout_passes=False)

_GDN = lax.GatherDimensionNumbers(
    offset_dims=(), collapsed_slice_dims=(0,), start_index_map=(0,))


def _vtake(vec, idx):
  return lax.gather(vec, idx[:, None], _GDN, slice_sizes=(1,),
                    mode=lax.GatherScatterMode.PROMISE_IN_BOUNDS)


@functools.partial(
    pl.kernel,
    out_type=(
        jax.ShapeDtypeStruct((_N_ENT, _D), jnp.float32),
        jax.ShapeDtypeStruct((_N_USR, _D), jnp.float32),
    ),
    mesh=_mesh,
    compiler_params=_cp,
    scratch_types=[
        pltpu.VMEM_SHARED((_N_ENT, _D), jnp.float32),
        pltpu.VMEM_SHARED((_N_ENT,), jnp.float32),
        pltpu.VMEM((_EB, _D), jnp.float32),
        pltpu.VMEM((_EB, _D), jnp.float32),
        pltpu.VMEM((_EB, _D), jnp.float32),
        pltpu.VMEM((_EB, _D), jnp.float32),
        pltpu.VMEM((_EB,), jnp.int32),
        pltpu.VMEM((_EB,), jnp.int32),
        pltpu.VMEM((_EB,), jnp.int32),
        pltpu.VMEM((_EB,), jnp.int32),
        pltpu.VMEM((_EB,), jnp.int32),
        pltpu.VMEM((_EB,), jnp.int32),
        pltpu.VMEM((_EB,), jnp.int32),
        pltpu.VMEM((_EB,), jnp.int32),
        pltpu.VMEM((_EB,), jnp.int32),
        pltpu.VMEM((_EB,), jnp.int32),
        pltpu.VMEM((_EB,), jnp.int32),
        pltpu.VMEM((_EB,), jnp.int32),
        pltpu.VMEM((_EB,), jnp.float32),
        pltpu.VMEM((_EB,), jnp.float32),
    ] + [pltpu.SemaphoreType.DMA] * 12,
)
def _sc_agg(ent_hbm, tbl_hbm, head_hbm, tail_hbm, rel_hbm, urow_hbm,
            ucol_hbm, uval_hbm,
            eagg_hbm, usum_hbm,
            sums, cnt, d0, d1, d2, d3, t0, t1, t2, t3, h0, h1, h2, h3,
            a0, a1, a2, a3, ones_v, cz_v,
            si0, si1, si2, si3, sg0, sg1, sg2, sg3, ss0, ss1, ss2, ss3):
  tid = lax.axis_index("subcore")
  cid = lax.axis_index("core")
  r0 = tid * _RPT
  ds_ = (d0, d1, d2, d3)
  ts_ = (t0, t1, t2, t3)
  hs_ = (h0, h1, h2, h3)
  as_ = (a0, a1, a2, a3)
  isem = (si0, si1, si2, si3)
  gsem = (sg0, sg1, sg2, sg3)
  ssem = (ss0, ss1, ss2, ss3)

  @pl.loop(0, _EB, step=16)
  def _(i):
    cz_v[pl.ds(i, 16)] = jnp.zeros((16,), jnp.float32)
    ones_v[pl.ds(i, 16)] = jnp.ones((16,), jnp.float32)

  @pl.loop(0, _EB)
  def _(i):
    for k in range(_D // 16):
      d0[i, pl.ds(16 * k, 16)] = jnp.zeros((16,), jnp.float32)

  @pl.loop(0, 560, step=80)
  def _(c):
    pltpu.async_copy(d0, sums.at[pl.ds(r0 + c, _EB)], sg0)

  pltpu.async_copy(d0.at[pl.ds(0, 64)], sums.at[pl.ds(r0 + 560, 64)], sg1)

  @pl.when(cid == 0)
  def _zcnt():
    @pl.loop(0, 560, step=80)
    def _(c):
      pltpu.async_copy(cz_v, cnt.at[pl.ds(r0 + c, _EB)], sg2)

    pltpu.async_copy(cz_v.at[pl.ds(0, 64)], cnt.at[pl.ds(r0 + 560, 64)], sg3)

    @pl.when(tid == _NT - 1)
    def _():
      pltpu.sync_copy(cz_v.at[pl.ds(0, _TAILN)], cnt.at[pl.ds(_TAIL0, _TAILN)])

  @pl.when(tid == _NT - 1)
  def _():
    pltpu.sync_copy(d0.at[pl.ds(0, _TAILN)], sums.at[pl.ds(_TAIL0, _TAILN)])

  @pl.loop(0, 560, step=80)
  def _(c):
    pltpu.make_async_copy(d0, sums.at[pl.ds(r0 + c, _EB)], sg0).wait()

  pltpu.make_async_copy(d0.at[pl.ds(0, 64)], sums.at[pl.ds(r0 + 560, 64)],
                        sg1).wait()

  @pl.when(cid == 0)
  def _zcnt_wait():
    @pl.loop(0, 560, step=80)
    def _(c):
      pltpu.make_async_copy(cz_v, cnt.at[pl.ds(r0 + c, _EB)], sg2).wait()

    pltpu.make_async_copy(cz_v.at[pl.ds(0, 64)], cnt.at[pl.ds(r0 + 560, 64)],
                          sg3).wait()

  plsc.subcore_barrier()

  def _pump(nb, src_hbm, gidx_hbm, sidx_hbm, aux_hbm, base, mul_fn, xform_fn,
            with_cnt):
    def _issue_idx(b, u):
      off = pl.ds(base + b * _EB, _EB)
      pltpu.async_copy(gidx_hbm.at[off], ts_[u], isem[u])
      pltpu.async_copy(sidx_hbm.at[off], hs_[u], isem[u])
      if aux_hbm is not None:
        pltpu.async_copy(aux_hbm.at[off], as_[u], isem[u])

    def _wait_idx(b, u):
      off = pl.ds(base + b * _EB, _EB)
      pltpu.make_async_copy(gidx_hbm.at[off], ts_[u], isem[u]).wait()
      pltpu.make_async_copy(sidx_hbm.at[off], hs_[u], isem[u]).wait()
      if aux_hbm is not None:
        pltpu.make_async_copy(aux_hbm.at[off], as_[u], isem[u]).wait()

    def _issue_gather(u):
      if xform_fn is not None:
        xform_fn(u)
      pltpu.async_copy(src_hbm.at[ts_[u]], ds_[u], gsem[u])

    def _wait_gather(u):
      pltpu.make_async_copy(src_hbm.at[ts_[u]], ds_[u], gsem[u]).wait()

    def _issue_scat(u):
      pltpu.async_copy(ds_[u], sums.at[hs_[u]], ssem[u], add=True)
      if with_cnt:
        pltpu.async_copy(ones_v, cnt.at[hs_[u]], ssem[u], add=True)

    def _wait_scat(u):
      pltpu.make_async_copy(ds_[u], sums.at[hs_[u]], ssem[u]).wait()
      if with_cnt:
        pltpu.make_async_copy(ones_v, cnt.at[hs_[u]], ssem[u]).wait()

    def _maybe(cond, fn):
      if isinstance(cond, bool):
        if cond:
          fn()
      else:
        @pl.when(cond)
        def _():
          fn()

    def _section(b, u):
      _maybe(b + 1 < nb, lambda: _wait_idx(b + 1, (u + 1) % 4))
      _maybe(b >= 2, lambda: _wait_scat((u + 2) % 4))
      _maybe(b + 1 < nb, lambda: _issue_gather((u + 1) % 4))
      _maybe(b + 2 < nb, lambda: _issue_idx(b + 2, (u + 2) % 4))
      _wait_gather(u)
      if mul_fn is not None:
        mul_fn(ds_[u], as_[u], b)
      _issue_scat(u)

    _issue_idx(0, 0)
    _issue_idx(1, 1)
    _wait_idx(0, 0)
    _issue_gather(0)

    nb4 = nb - (nb % 4)

    @pl.loop(0, nb4, step=4)
    def _(b0):
      for u in range(4):
        _section(b0 + u, u)

    for t in range(nb % 4):
      _section(nb4 + t, t)

    _wait_scat((nb - 2) % 4)
    _wait_scat((nb - 1) % 4)

  def _xform_edges(u):
    @pl.loop(0, _EB, step=16)
    def _(i):
      ts_[u][pl.ds(i, 16)] = (as_[u][pl.ds(i, 16)] * _N_ENT
                              + ts_[u][pl.ds(i, 16)])

  def _mul_users(dbuf, abuf, b):
    @pl.loop(0, _EB, step=16)
    def _(g):
      valv = plsc.bitcast(abuf[pl.ds(g, 16)], jnp.float32)
      for j in range(16):
        vb = _vtake(valv, jnp.full((16,), j, jnp.int32))
        e = g + j
        for k in range(_D // 16):
          dbuf[e, pl.ds(16 * k, 16)] = dbuf[e, pl.ds(16 * k, 16)] * vb

  @pl.when(cid == 0)
  def _edges():
    _pump(_NB_E, tbl_hbm, tail_hbm, head_hbm, rel_hbm,
          tid * _EDGES_PER_TILE, None, _xform_edges, with_cnt=True)

  @pl.when(cid == 1)
  def _users():
    _pump(_NB_U, ent_hbm, ucol_hbm, urow_hbm, uval_hbm,
          tid * _NNZ_PER_TILE, _mul_users, None, with_cnt=False)

  plsc.subcore_barrier()

  def _divide(dbuf, n):
    @pl.loop(0, n, step=16)
    def _(g):
      cv = cz_v[pl.ds(g, 16)]
      rv = 1.0 / jnp.maximum(cv, 1.0)
      for j in range(16):
        sb = _vtake(rv, jnp.full((16,), j, jnp.int32))
        e = g + j
        for k in range(_D // 16):
          dbuf[e, pl.ds(16 * k, 16)] = dbuf[e, pl.ds(16 * k, 16)] * sb

  def _copy_out(dst_hbm, divide):
    def _chunk(c, n, u):
      pltpu.sync_copy(sums.at[pl.ds(r0 + c, n)], ds_[u].at[pl.ds(0, n)])
      if divide:
        pltpu.sync_copy(cnt.at[pl.ds(r0 + c, n)], cz_v.at[pl.ds(0, n)])
        _divide(ds_[u], n)
      pltpu.async_copy(ds_[u].at[pl.ds(0, n)], dst_hbm.at[pl.ds(r0 + c, n)],
                       gsem[u])

    for ci in range(8):
      u = ci % 4
      n = _EB if ci < 7 else 64
      if ci >= 4:
        pltpu.make_async_copy(ds_[u].at[pl.ds(0, _EB)],
                              dst_hbm.at[pl.ds(r0 + (ci - 4) * _EB, _EB)],
                              gsem[u]).wait()
      _chunk(ci * _EB, n, u)

    for ci in range(4, 8):
      u = ci % 4
      n = _EB if ci < 7 else 64
      pltpu.make_async_copy(ds_[u].at[pl.ds(0, n)],
                            dst_hbm.at[pl.ds(r0 + ci * _EB, n)],
                            gsem[u]).wait()

    @pl.when(tid == _NT - 1)
    def _():
      pltpu.sync_copy(sums.at[pl.ds(_TAIL0, _TAILN)],
                      ds_[0].at[pl.ds(0, _TAILN)])
      if divide:
        pltpu.sync_copy(cnt.at[pl.ds(_TAIL0, _TAILN)],
                        cz_v.at[pl.ds(0, _TAILN)])
        _divide(ds_[0], _TAILN)
      pltpu.sync_copy(ds_[0].at[pl.ds(0, _TAILN)],
                      dst_hbm.at[pl.ds(_TAIL0, _TAILN)])

  @pl.when(cid == 0)
  def _out_e():
    _copy_out(eagg_hbm, divide=True)

  @pl.when(cid == 1)
  def _out_u():
    _copy_out(usum_hbm, divide=False)


def _premul_body(ent_ref, w_ref, out_ref):
  r = pl.program_id(0)
  out_ref[...] = ent_ref[...] * w_ref[pl.ds(r, 1), :]


def _premul(entity_emb, weight):
  nb = _N_ENT // _BLK
  return pl.pallas_call(
      _premul_body,
      grid=(_NREL, nb),
      in_specs=[
          pl.BlockSpec((_BLK, _D), lambda r, i: (i, 0)),
          pl.BlockSpec((_NREL, _D), lambda r, i: (0, 0)),
      ],
      out_specs=pl.BlockSpec((_BLK, _D), lambda r, i: (r * nb + i, 0)),
      out_shape=jax.ShapeDtypeStruct((_NREL * _N_ENT, _D), jnp.float32),
  )(entity_emb, weight)


def _finish_body(usum_ref, user_ref, lat_ref, w_ref, datt_ref, uout_ref):
  logits = lax.dot_general(
      user_ref[...], lat_ref[...], (((1,), (1,)), ((), ())),
      precision=lax.Precision.HIGHEST, preferred_element_type=jnp.float32)
  score = jax.nn.softmax(logits, axis=1)
  dw = jax.nn.softmax(datt_ref[...], axis=-1)
  dw2 = lax.dot_general(
      dw, w_ref[...], (((1,), (0,)), ((), ())),
      precision=lax.Precision.HIGHEST, preferred_element_type=jnp.float32)
  mod = lax.dot_general(
      score, dw2, (((1,), (0,)), ((), ())),
      precision=lax.Precision.HIGHEST, preferred_element_type=jnp.float32)
  uout_ref[...] = usum_ref[...] * (1.0 + mod)


_BLK = 1000


def _finish(usum, user_emb, latent_emb, weight, disen_weight_att):
  n_blocks = _N_USR // _BLK
  return pl.pallas_call(
      _finish_body,
      grid=(n_blocks,),
      in_specs=[
          pl.BlockSpec((_BLK, _D), lambda i: (i, 0)),
          pl.BlockSpec((_BLK, _D), lambda i: (i, 0)),
          pl.BlockSpec((4, _D), lambda i: (0, 0)),
          pl.BlockSpec((_NREL, _D), lambda i: (0, 0)),
          pl.BlockSpec((4, _NREL), lambda i: (0, 0)),
      ],
      out_specs=pl.BlockSpec((_BLK, _D), lambda i: (i, 0)),
      out_shape=jax.ShapeDtypeStruct((_N_USR, _D), jnp.float32),
  )(usum, user_emb, latent_emb, weight, disen_weight_att)


def kernel(entity_emb, user_emb, latent_emb, edge_index, edge_type,
           interact_idx, interact_val, weight, disen_weight_att):
  head = edge_index[0].astype(jnp.int32)
  tail = edge_index[1].astype(jnp.int32)
  rel = ((edge_type.astype(jnp.int32) - 1) % _NREL).astype(jnp.int32)
  urow = interact_idx[0].astype(jnp.int32)
  ucol = interact_idx[1].astype(jnp.int32)
  uval_i = lax.bitcast_convert_type(interact_val, jnp.int32)
  tbl = _premul(entity_emb, weight)
  eagg, usum = _sc_agg(entity_emb, tbl, head, tail, rel, urow, ucol,
                       uval_i)
  user_agg = _finish(usum, user_emb, latent_emb, weight, disen_weight_att)
  return (eagg, user_agg)
